# baseline (speedup 1.0000x reference)
import functools

import jax
import jax.numpy as jnp
from jax import lax
from jax.experimental import pallas as pl
from jax.experimental.pallas import tpu as pltpu
from jax.experimental.pallas import tpu_sc as plsc

_VOCAB = 100000
_EMBED = 64
_BATCH = 16384
_L = 4

_NC = 2
_NS = 16
_NW = _NC * _NS
_PW = _BATCH // _NW
_CH = 128
_NCHUNK = _PW // _CH
_RW = _CH * _L
_ZROW = _RW

_L1P = (
    0.9999995186059871, -0.4999635658266627, 0.3326525584507218,
    -0.24453388974691162, 0.17659864777924483, -0.10679931294111752,
    0.043659288935938435, -0.008466410428720516,
)


def _log_sigmoid(x):
    na = -jnp.abs(x)
    e = jnp.exp(na)
    q = jnp.full((16,), _L1P[-1], jnp.float32)
    for coef in _L1P[-2::-1]:
        q = q * e + jnp.float32(coef)
    return jnp.minimum(x, 0.0) - e * q


def _compute_chunk(c, idx_flat, rows, outb, iota4):
    iota17 = lax.iota(jnp.int32, 16) * 17

    @plsc.parallel_loop(0, _CH // 16)
    def group_body(g):
        base = g * 64
        cj = [
            plsc.load_gather(idx_flat, [iota4 + (c * 512 + base + j)])
            for j in range(_L)
        ]
        p = [iota4 + (base + j) for j in range(_L)]
        ridx = [
            jnp.where(cj[j] == 0, jnp.int32(_ZROW), p[j]) for j in range(_L)
        ]

        zero = jnp.zeros((16,), jnp.float32)

        @plsc.parallel_loop(0, _EMBED, unroll=8, carry=(zero,) * 8)
        def k_loop(k, carry):
            a0, a1, a2, a3, n0, n1, n2, n3 = carry
            colv = (jnp.full((16,), k, jnp.int32) + iota17) & 63
            v = [plsc.load_gather(rows, [ridx[j], colv]) for j in range(_L)]
            s = (v[0] + v[1]) + (v[2] + v[3])
            return (
                a0 + v[0] * s, a1 + v[1] * s, a2 + v[2] * s, a3 + v[3] * s,
                n0 + v[0] * v[0], n1 + v[1] * v[1],
                n2 + v[2] * v[2], n3 + v[3] * v[3],
            )

        accs = k_loop[:4]
        nns = k_loop[4:]

        one = jnp.float32(1.0)
        zf = jnp.float32(0.0)
        third = jnp.float32(1.0 / 3.0)
        for i in range(_L):
            mf = jnp.zeros((16,), jnp.float32)
            for j in range(_L):
                mf = mf + jnp.where(cj[j] == cj[i], one, zf)
            score = (accs[i] - mf * nns[i]) * third
            zi = jnp.zeros((16,), jnp.int32)
            plsc.store_scatter(outb, [p[i], zi, zi], _log_sigmoid(score))


def _sc_body(chf_hbm, table_hbm, out_hbm, idx_flat,
             rows0, rows1, out0, out1, sem0, sem1):
    wid = lax.axis_index("s") * _NC + lax.axis_index("c")
    iota4 = lax.iota(jnp.int32, 16) * 4

    pltpu.sync_copy(chf_hbm.at[pl.ds(wid * 2048, 2048)], idx_flat)

    rows_bufs = (rows0, rows1)
    out_bufs = (out0, out1)
    sems = (sem0, sem1)
    zero16 = jnp.zeros((16,), jnp.float32)
    for b in range(2):
        for kk in range(_EMBED // 16):
            rows_bufs[b][_ZROW, pl.ds(kk * 16, 16)] = zero16

    def fire(c, b):
        return [
            pltpu.async_copy(
                table_hbm.at[idx_flat.at[pl.ds(c * 512 + i * 128, 128)]],
                rows_bufs[b].at[pl.ds(i * 128, 128)],
                sems[b],
            )
            for i in range(4)
        ]

    copies = fire(0, 0)
    for c in range(_NCHUNK):
        b = c & 1
        if c + 1 < _NCHUNK:
            next_copies = fire(c + 1, 1 - b)
        for cp in copies:
            cp.wait()
        _compute_chunk(c, idx_flat, rows_bufs[b], out_bufs[b], iota4)
        pltpu.sync_copy(
            out_bufs[b], out_hbm.at[pl.ds(wid * (_PW * _L) + c * _RW, _RW)]
        )
        if c + 1 < _NCHUNK:
            copies = next_copies


def kernel(chords, table):
    chf = chords.reshape(_BATCH * _L)
    mesh = plsc.VectorSubcoreMesh(
        core_axis_name="c", subcore_axis_name="s",
        num_cores=_NC, num_subcores=_NS,
    )
    scores = pl.kernel(
        _sc_body,
        out_type=jax.ShapeDtypeStruct((_BATCH * _L, 1, 1), jnp.float32),
        mesh=mesh,
        compiler_params=pltpu.CompilerParams(
            needs_layout_passes=False, use_tc_tiling_on_sc=False
        ),
        scratch_types=[
            pltpu.VMEM((_PW * _L,), jnp.int32),
            pltpu.VMEM((_RW + 8, _EMBED), jnp.float32),
            pltpu.VMEM((_RW + 8, _EMBED), jnp.float32),
            pltpu.VMEM((_RW, 1, 1), jnp.float32),
            pltpu.VMEM((_RW, 1, 1), jnp.float32),
            pltpu.SemaphoreType.DMA,
            pltpu.SemaphoreType.DMA,
        ],
    )(chf, table)
    return scores


# speedup vs baseline: 2.5187x; 2.5187x over previous
import functools

import jax
import jax.numpy as jnp
from jax import lax
from jax.experimental import pallas as pl
from jax.experimental.pallas import tpu as pltpu
from jax.experimental.pallas import tpu_sc as plsc

_VOCAB = 100000
_EMBED = 64
_BATCH = 16384
_L = 4

_NC = 2
_NS = 16
_NW = _NC * _NS
_PW = _BATCH // _NW
_CH = 128
_NCHUNK = _PW // _CH
_RW = _CH * _L
_ZROW = _RW

_L1P = (
    0.9999995186059871, -0.4999635658266627, 0.3326525584507218,
    -0.24453388974691162, 0.17659864777924483, -0.10679931294111752,
    0.043659288935938435, -0.008466410428720516,
)


def _log_sigmoid(x):
    na = -jnp.abs(x)
    e = jnp.exp(na)
    q = jnp.full((16,), _L1P[-1], jnp.float32)
    for coef in _L1P[-2::-1]:
        q = q * e + jnp.float32(coef)
    return jnp.minimum(x, 0.0) - e * q


def _compute_chunk(c, idx_flat, rows, outb, iota4):
    iota17 = lax.iota(jnp.int32, 16) * 17

    @plsc.parallel_loop(0, _CH // 16)
    def group_body(g):
        base = g * 64
        cj = [
            plsc.load_gather(idx_flat, [iota4 + (c * 512 + base + j)])
            for j in range(_L)
        ]
        p = [iota4 + (base + j) for j in range(_L)]
        ridx = [
            jnp.where(cj[j] == 0, jnp.int32(_ZROW), p[j]) for j in range(_L)
        ]

        zero = jnp.zeros((16,), jnp.float32)

        @plsc.parallel_loop(0, _EMBED, unroll=8, carry=(zero,) * 8)
        def k_loop(k, carry):
            a0, a1, a2, a3, n0, n1, n2, n3 = carry
            colv = (jnp.full((16,), k, jnp.int32) + iota17) & 63
            v = [plsc.load_gather(rows, [ridx[j], colv]) for j in range(_L)]
            s = (v[0] + v[1]) + (v[2] + v[3])
            return (
                a0 + v[0] * s, a1 + v[1] * s, a2 + v[2] * s, a3 + v[3] * s,
                n0 + v[0] * v[0], n1 + v[1] * v[1],
                n2 + v[2] * v[2], n3 + v[3] * v[3],
            )

        accs = k_loop[:4]
        nns = k_loop[4:]

        one = jnp.float32(1.0)
        zf = jnp.float32(0.0)
        third = jnp.float32(1.0 / 3.0)
        for i in range(_L):
            mf = jnp.zeros((16,), jnp.float32)
            for j in range(_L):
                mf = mf + jnp.where(cj[j] == cj[i], one, zf)
            score = (accs[i] - mf * nns[i]) * third
            plsc.store_scatter(outb, [p[i]], _log_sigmoid(score))


def _sc_body(chf_hbm, table_hbm, out_hbm, idx_flat,
             rows0, rows1, out0, out1, sem0, sem1):
    wid = lax.axis_index("s") * _NC + lax.axis_index("c")
    iota4 = lax.iota(jnp.int32, 16) * 4

    pltpu.sync_copy(chf_hbm.at[pl.ds(wid * 2048, 2048)], idx_flat)

    rows_bufs = (rows0, rows1)
    out_bufs = (out0, out1)
    sems = (sem0, sem1)
    zero16 = jnp.zeros((16,), jnp.float32)
    for b in range(2):
        for kk in range(_EMBED // 16):
            rows_bufs[b][_ZROW, pl.ds(kk * 16, 16)] = zero16

    def fire(c, b):
        return [
            pltpu.async_copy(
                table_hbm.at[idx_flat.at[pl.ds(c * 512 + i * 128, 128)]],
                rows_bufs[b].at[pl.ds(i * 128, 128)],
                sems[b],
            )
            for i in range(4)
        ]

    copies = fire(0, 0)
    for c in range(_NCHUNK):
        b = c & 1
        if c + 1 < _NCHUNK:
            next_copies = fire(c + 1, 1 - b)
        for cp in copies:
            cp.wait()
        _compute_chunk(c, idx_flat, rows_bufs[b], out_bufs[b], iota4)
        pltpu.sync_copy(
            out_bufs[b], out_hbm.at[pl.ds(wid * (_PW * _L) + c * _RW, _RW)]
        )
        if c + 1 < _NCHUNK:
            copies = next_copies


def kernel(chords, table):
    chf = chords.reshape(_BATCH * _L)
    mesh = plsc.VectorSubcoreMesh(
        core_axis_name="c", subcore_axis_name="s",
        num_cores=_NC, num_subcores=_NS,
    )
    scores = pl.kernel(
        _sc_body,
        out_type=jax.ShapeDtypeStruct((_BATCH * _L,), jnp.float32),
        mesh=mesh,
        compiler_params=pltpu.CompilerParams(
            needs_layout_passes=False, use_tc_tiling_on_sc=False
        ),
        scratch_types=[
            pltpu.VMEM((_PW * _L,), jnp.int32),
            pltpu.VMEM((_RW + 8, _EMBED), jnp.float32),
            pltpu.VMEM((_RW + 8, _EMBED), jnp.float32),
            pltpu.VMEM((_RW,), jnp.float32),
            pltpu.VMEM((_RW,), jnp.float32),
            pltpu.SemaphoreType.DMA,
            pltpu.SemaphoreType.DMA,
        ],
    )(chf, table)
    return scores.reshape(_BATCH * _L, 1, 1)
